# Initial kernel scaffold; baseline (speedup 1.0000x reference)
import functools

import jax
import jax.numpy as jnp
from jax import lax
from jax.experimental import pallas as pl
from jax.experimental.pallas import tpu as pltpu, tpu_sc as plsc

N = 10000
E = 320000
F_IN = 128
H = 64

NC = 2
NS = 16
NW = NC * NS
CHUNK = 128
CPW = -(-E // (NW * CHUNK))
EPW = CPW * CHUNK
EP = NW * EPW
NP = 10112
RPT = NP // NS
YRPT = N // NS


def _mm2_body(x_ref, wl_ref, wr_ref, y_ref, z_ref):
    xb = x_ref[...]
    y_ref[...] = jnp.dot(xb, wl_ref[...], preferred_element_type=jnp.float32)
    z_ref[...] = jnp.dot(xb, wr_ref[...], preferred_element_type=jnp.float32)


def _dual_matmul(x, wl_t, wr_t):
    blk = 1000
    grid = N // blk
    return pl.pallas_call(
        _mm2_body,
        grid=(grid,),
        in_specs=[
            pl.BlockSpec((blk, F_IN), lambda i: (i, 0)),
            pl.BlockSpec((F_IN, H), lambda i: (0, 0)),
            pl.BlockSpec((F_IN, H), lambda i: (0, 0)),
        ],
        out_specs=[
            pl.BlockSpec((blk, H), lambda i: (i, 0)),
            pl.BlockSpec((blk, H), lambda i: (i, 0)),
        ],
        out_shape=[
            jax.ShapeDtypeStruct((N, H), jnp.float32),
            jax.ShapeDtypeStruct((N, H), jnp.float32),
        ],
    )(x, wl_t, wr_t)


def _sc_body(y_hbm, src_hbm, dst_hbm, za_hbm, zc_hbm,
             agg_out, cnt_out,
             y_sh, agg_sh, cnt_sh,
             src_v, dst_v, rows_v, ones_v):
    c = lax.axis_index("c")
    s = lax.axis_index("s")
    w = c * NS + s

    pltpu.sync_copy(za_hbm.at[pl.ds(s * RPT, RPT)], agg_sh.at[pl.ds(s * RPT, RPT)])
    pltpu.sync_copy(zc_hbm.at[pl.ds(s * RPT, RPT)], cnt_sh.at[pl.ds(s * RPT, RPT)])
    pltpu.sync_copy(y_hbm.at[pl.ds(s * YRPT, YRPT)], y_sh.at[pl.ds(s * YRPT, YRPT)])

    pltpu.sync_copy(src_hbm.at[w], src_v)
    pltpu.sync_copy(dst_hbm.at[w], dst_v)

    for i in range(CHUNK // 16):
        ones_v[pl.ds(i * 16, 16)] = jnp.ones((16,), jnp.float32)

    plsc.subcore_barrier()

    def chunk_step(j, carry):
        pltpu.sync_copy(y_sh.at[src_v.at[j]], rows_v)
        pltpu.sync_copy(rows_v, agg_sh.at[dst_v.at[j]], add=True)
        pltpu.sync_copy(ones_v, cnt_sh.at[dst_v.at[j]], add=True)
        return carry

    lax.fori_loop(0, CPW, chunk_step, 0)

    plsc.subcore_barrier()

    pltpu.sync_copy(agg_sh.at[pl.ds(s * RPT, RPT)],
                    agg_out.at[c].at[pl.ds(s * RPT, RPT)])
    pltpu.sync_copy(cnt_sh.at[pl.ds(s * RPT, RPT)],
                    cnt_out.at[c].at[pl.ds(s * RPT, RPT)])


_sc_aggregate = pl.kernel(
    _sc_body,
    out_type=[
        jax.ShapeDtypeStruct((NC, NP, H), jnp.float32),
        jax.ShapeDtypeStruct((NC, NP), jnp.float32),
    ],
    mesh=plsc.VectorSubcoreMesh(core_axis_name="c", subcore_axis_name="s"),
    scratch_types=[
        pltpu.VMEM_SHARED((N, H), jnp.float32),
        pltpu.VMEM_SHARED((NP, H), jnp.float32),
        pltpu.VMEM_SHARED((NP,), jnp.float32),
        pltpu.VMEM((CPW, CHUNK), jnp.int32),
        pltpu.VMEM((CPW, CHUNK), jnp.int32),
        pltpu.VMEM((CHUNK, H), jnp.float32),
        pltpu.VMEM((CHUNK,), jnp.float32),
    ],
)


def _fin_body(agg_ref, cnt_ref, z_ref, b_ref, o_ref):
    a = agg_ref[0] + agg_ref[1]
    cnt = jnp.maximum(cnt_ref[0] + cnt_ref[1], 1.0)
    o_ref[...] = a / cnt[:, None] + z_ref[...] + b_ref[...]


def _finish(agg_p, cnt_p, z, b_row):
    blk = 1000
    grid = N // blk
    return pl.pallas_call(
        _fin_body,
        grid=(grid,),
        in_specs=[
            pl.BlockSpec((NC, blk, H), lambda i: (0, i, 0)),
            pl.BlockSpec((NC, blk), lambda i: (0, i)),
            pl.BlockSpec((blk, H), lambda i: (i, 0)),
            pl.BlockSpec((1, H), lambda i: (0, 0)),
        ],
        out_specs=pl.BlockSpec((blk, H), lambda i: (i, 0)),
        out_shape=jax.ShapeDtypeStruct((N, H), jnp.float32),
    )(agg_p, cnt_p, z, b_l_row := b_row)


def kernel(x, edge_index, W_l, W_r, b_l):
    y, z = _dual_matmul(x, W_l.T, W_r.T)

    src = edge_index[0].astype(jnp.int32)
    dst = edge_index[1].astype(jnp.int32)
    pad = EP - E
    src_p = jnp.concatenate([src, jnp.zeros((pad,), jnp.int32)]).reshape(NW, CPW, CHUNK)
    dst_p = jnp.concatenate([dst, jnp.full((pad,), N, jnp.int32)]).reshape(NW, CPW, CHUNK)

    za = jnp.zeros((NP, H), jnp.float32)
    zc = jnp.zeros((NP,), jnp.float32)

    agg_p, cnt_p = _sc_aggregate(y, src_p, dst_p, za, zc)

    return _finish(agg_p, cnt_p, z, b_l.reshape(1, H))


# speedup vs baseline: 10.4717x; 10.4717x over previous
import functools

import jax
import jax.numpy as jnp
from jax import lax
from jax.experimental import pallas as pl
from jax.experimental.pallas import tpu as pltpu, tpu_sc as plsc

N = 10000
E = 320000
F_IN = 128
H = 64

NC = 2
NS = 16
NW = NC * NS
CHUNK = 128
CPW = -(-E // (NW * CHUNK))
EPW = CPW * CHUNK
EP = NW * EPW
NP = 10112
RPT = NP // NS


def _mm2_body(x_ref, wl_ref, wr_ref, y_ref, z_ref):
    xb = x_ref[...]
    y_ref[...] = jnp.dot(xb, wl_ref[...], preferred_element_type=jnp.float32)
    z_ref[...] = jnp.dot(xb, wr_ref[...], preferred_element_type=jnp.float32)


def _dual_matmul(x, wl_t, wr_t):
    blk = RPT
    grid = NP // blk
    return pl.pallas_call(
        _mm2_body,
        grid=(grid,),
        in_specs=[
            pl.BlockSpec((blk, F_IN), lambda i: (i, 0)),
            pl.BlockSpec((F_IN, H), lambda i: (0, 0)),
            pl.BlockSpec((F_IN, H), lambda i: (0, 0)),
        ],
        out_specs=[
            pl.BlockSpec((blk, H), lambda i: (i, 0)),
            pl.BlockSpec((blk, H), lambda i: (i, 0)),
        ],
        out_shape=[
            jax.ShapeDtypeStruct((NP, H), jnp.float32),
            jax.ShapeDtypeStruct((NP, H), jnp.float32),
        ],
    )(x, wl_t, wr_t)


def _sc_body(y_hbm, src_hbm, dst_hbm, za_hbm, zc_hbm,
             agg_out, cnt_out,
             y_sh, agg_sh, cnt_sh,
             src_v, dst_v, rows_v, ones_v):
    c = lax.axis_index("c")
    s = lax.axis_index("s")
    w = c * NS + s

    pltpu.sync_copy(za_hbm.at[pl.ds(s * RPT, RPT)], agg_sh.at[pl.ds(s * RPT, RPT)])
    pltpu.sync_copy(zc_hbm.at[pl.ds(s * RPT, RPT)], cnt_sh.at[pl.ds(s * RPT, RPT)])
    pltpu.sync_copy(y_hbm.at[pl.ds(s * RPT, RPT)], y_sh.at[pl.ds(s * RPT, RPT)])

    pltpu.sync_copy(src_hbm.at[w], src_v)
    pltpu.sync_copy(dst_hbm.at[w], dst_v)

    def ones_init(j, carry):
        ones_v[j] = jnp.ones((16,), jnp.float32)
        return carry

    lax.fori_loop(0, CHUNK, ones_init, 0)

    plsc.subcore_barrier()

    def chunk_step(j, carry):
        pltpu.sync_copy(y_sh.at[src_v.at[j]], rows_v)
        pltpu.sync_copy(rows_v, agg_sh.at[dst_v.at[j]], add=True)
        pltpu.sync_copy(ones_v, cnt_sh.at[dst_v.at[j]], add=True)
        return carry

    lax.fori_loop(0, CPW, chunk_step, 0)

    plsc.subcore_barrier()

    pltpu.sync_copy(agg_sh.at[pl.ds(s * RPT, RPT)],
                    agg_out.at[c].at[pl.ds(s * RPT, RPT)])
    pltpu.sync_copy(cnt_sh.at[pl.ds(s * RPT, RPT)],
                    cnt_out.at[c].at[pl.ds(s * RPT, RPT)])


_sc_aggregate = pl.kernel(
    _sc_body,
    out_type=[
        jax.ShapeDtypeStruct((NC, NP, H), jnp.float32),
        jax.ShapeDtypeStruct((NC, NP, 16), jnp.float32),
    ],
    mesh=plsc.VectorSubcoreMesh(core_axis_name="c", subcore_axis_name="s"),
    compiler_params=pltpu.CompilerParams(use_tc_tiling_on_sc=False),
    scratch_types=[
        pltpu.VMEM_SHARED((NP, H), jnp.float32),
        pltpu.VMEM_SHARED((NP, H), jnp.float32),
        pltpu.VMEM_SHARED((NP, 16), jnp.float32),
        pltpu.VMEM((CPW, CHUNK), jnp.int32),
        pltpu.VMEM((CPW, CHUNK), jnp.int32),
        pltpu.VMEM((CHUNK, H), jnp.float32),
        pltpu.VMEM((CHUNK, 16), jnp.float32),
    ],
)


def _fin_body(agg_ref, cnt_ref, z_ref, b_ref, o_ref):
    a = agg_ref[0] + agg_ref[1]
    cnt = jnp.maximum(cnt_ref[0][:, :1] + cnt_ref[1][:, :1], 1.0)
    o_ref[...] = a / cnt + z_ref[...] + b_ref[...]


def _finish(agg_p, cnt_p, z, b_row):
    blk = 1000
    grid = N // blk
    return pl.pallas_call(
        _fin_body,
        grid=(grid,),
        in_specs=[
            pl.BlockSpec((NC, blk, H), lambda i: (0, i, 0)),
            pl.BlockSpec((NC, blk, 16), lambda i: (0, i, 0)),
            pl.BlockSpec((blk, H), lambda i: (i, 0)),
            pl.BlockSpec((1, H), lambda i: (0, 0)),
        ],
        out_specs=pl.BlockSpec((blk, H), lambda i: (i, 0)),
        out_shape=jax.ShapeDtypeStruct((N, H), jnp.float32),
    )(agg_p, cnt_p, z, b_row)


def kernel(x, edge_index, W_l, W_r, b_l):
    x_p = jnp.concatenate([x, jnp.zeros((NP - N, F_IN), jnp.float32)])
    y, z = _dual_matmul(x_p, W_l.T, W_r.T)

    src = edge_index[0].astype(jnp.int32)
    dst = edge_index[1].astype(jnp.int32)
    pad = EP - E
    src_p = jnp.concatenate([src, jnp.zeros((pad,), jnp.int32)]).reshape(NW, CPW, CHUNK)
    dst_p = jnp.concatenate([dst, jnp.full((pad,), N, jnp.int32)]).reshape(NW, CPW, CHUNK)

    za = jnp.zeros((NP, H), jnp.float32)
    zc = jnp.zeros((NP, 16), jnp.float32)

    agg_p, cnt_p = _sc_aggregate(y, src_p, dst_p, za, zc)

    return _finish(agg_p, cnt_p, z, b_l.reshape(1, H))


# speedup vs baseline: 11.9232x; 1.1386x over previous
import functools

import jax
import jax.numpy as jnp
from jax import lax
from jax.experimental import pallas as pl
from jax.experimental.pallas import tpu as pltpu, tpu_sc as plsc

N = 10000
E = 320000
F_IN = 128
H = 64

NC = 2
NS = 16
NW = NC * NS
CHUNK = 128
CPW = -(-E // (NW * CHUNK))
EPW = CPW * CHUNK
EP = NW * EPW
NP = 10112
RPT = NP // NS


def _mm2_body(x_ref, wl_ref, wr_ref, y_ref, z_ref):
    xb = x_ref[...]
    y_ref[...] = jnp.dot(xb, wl_ref[...], preferred_element_type=jnp.float32)
    z_ref[...] = jnp.dot(xb, wr_ref[...], preferred_element_type=jnp.float32)


def _dual_matmul(x, wl_t, wr_t):
    blk = RPT
    grid = NP // blk
    return pl.pallas_call(
        _mm2_body,
        grid=(grid,),
        in_specs=[
            pl.BlockSpec((blk, F_IN), lambda i: (i, 0)),
            pl.BlockSpec((F_IN, H), lambda i: (0, 0)),
            pl.BlockSpec((F_IN, H), lambda i: (0, 0)),
        ],
        out_specs=[
            pl.BlockSpec((blk, H), lambda i: (i, 0)),
            pl.BlockSpec((blk, H), lambda i: (i, 0)),
        ],
        out_shape=[
            jax.ShapeDtypeStruct((NP, H), jnp.float32),
            jax.ShapeDtypeStruct((NP, H), jnp.float32),
        ],
    )(x, wl_t, wr_t)


def _sc_body(y_hbm, src_hbm, dst_hbm, za_hbm, zc_hbm,
             agg_out, cnt_out,
             y_sh, agg_sh, cnt_sh,
             src_v, dst_v, rows_v, ones_v,
             gsem, ssem, csem):
    c = lax.axis_index("c")
    s = lax.axis_index("s")
    w = c * NS + s

    pltpu.sync_copy(za_hbm.at[pl.ds(s * RPT, RPT)], agg_sh.at[pl.ds(s * RPT, RPT)])
    pltpu.sync_copy(zc_hbm.at[pl.ds(s * RPT, RPT)], cnt_sh.at[pl.ds(s * RPT, RPT)])
    pltpu.sync_copy(y_hbm.at[pl.ds(s * RPT, RPT)], y_sh.at[pl.ds(s * RPT, RPT)])

    pltpu.sync_copy(src_hbm.at[w], src_v)
    pltpu.sync_copy(dst_hbm.at[w], dst_v)

    def ones_init(j, carry):
        ones_v[j] = jnp.ones((16,), jnp.float32)
        return carry

    lax.fori_loop(0, CHUNK, ones_init, 0)

    plsc.subcore_barrier()

    pltpu.async_copy(y_sh.at[src_v.at[0]], rows_v.at[0], gsem)

    def chunk_step(j, carry):
        b = lax.rem(j, 2)
        nb = 1 - b

        pltpu.make_async_copy(y_sh.at[src_v.at[j]], rows_v.at[b], gsem).wait()

        @pl.when(j >= 1)
        def _wait_prev_scatter():
            pltpu.make_async_copy(rows_v.at[nb], agg_sh.at[dst_v.at[j]],
                                  ssem).wait()
            pltpu.make_async_copy(ones_v, cnt_sh.at[dst_v.at[j]], csem).wait()

        @pl.when(j + 1 < CPW)
        def _prefetch_next():
            pltpu.async_copy(y_sh.at[src_v.at[j + 1]], rows_v.at[nb], gsem)

        pltpu.async_copy(rows_v.at[b], agg_sh.at[dst_v.at[j]], ssem, add=True)
        pltpu.async_copy(ones_v, cnt_sh.at[dst_v.at[j]], csem, add=True)
        return carry

    lax.fori_loop(0, CPW, chunk_step, 0)

    last = CPW - 1
    pltpu.make_async_copy(rows_v.at[lax.rem(last, 2)],
                          agg_sh.at[dst_v.at[last]], ssem).wait()
    pltpu.make_async_copy(ones_v, cnt_sh.at[dst_v.at[last]], csem).wait()

    plsc.subcore_barrier()

    pltpu.sync_copy(agg_sh.at[pl.ds(s * RPT, RPT)],
                    agg_out.at[c].at[pl.ds(s * RPT, RPT)])
    pltpu.sync_copy(cnt_sh.at[pl.ds(s * RPT, RPT)],
                    cnt_out.at[c].at[pl.ds(s * RPT, RPT)])


_sc_aggregate = pl.kernel(
    _sc_body,
    out_type=[
        jax.ShapeDtypeStruct((NC, NP, H), jnp.float32),
        jax.ShapeDtypeStruct((NC, NP, 16), jnp.float32),
    ],
    mesh=plsc.VectorSubcoreMesh(core_axis_name="c", subcore_axis_name="s"),
    compiler_params=pltpu.CompilerParams(use_tc_tiling_on_sc=False),
    scratch_types=[
        pltpu.VMEM_SHARED((NP, H), jnp.float32),
        pltpu.VMEM_SHARED((NP, H), jnp.float32),
        pltpu.VMEM_SHARED((NP, 16), jnp.float32),
        pltpu.VMEM((CPW, CHUNK), jnp.int32),
        pltpu.VMEM((CPW, CHUNK), jnp.int32),
        pltpu.VMEM((2, CHUNK, H), jnp.float32),
        pltpu.VMEM((CHUNK, 16), jnp.float32),
        pltpu.SemaphoreType.DMA,
        pltpu.SemaphoreType.DMA,
        pltpu.SemaphoreType.DMA,
    ],
)


def _fin_body(agg_ref, cnt_ref, z_ref, b_ref, o_ref):
    a = agg_ref[0] + agg_ref[1]
    cnt = jnp.maximum(cnt_ref[0][:, :1] + cnt_ref[1][:, :1], 1.0)
    o_ref[...] = a / cnt + z_ref[...] + b_ref[...]


def _finish(agg_p, cnt_p, z, b_row):
    blk = 1000
    grid = N // blk
    return pl.pallas_call(
        _fin_body,
        grid=(grid,),
        in_specs=[
            pl.BlockSpec((NC, blk, H), lambda i: (0, i, 0)),
            pl.BlockSpec((NC, blk, 16), lambda i: (0, i, 0)),
            pl.BlockSpec((blk, H), lambda i: (i, 0)),
            pl.BlockSpec((1, H), lambda i: (0, 0)),
        ],
        out_specs=pl.BlockSpec((blk, H), lambda i: (i, 0)),
        out_shape=jax.ShapeDtypeStruct((N, H), jnp.float32),
    )(agg_p, cnt_p, z, b_row)


def kernel(x, edge_index, W_l, W_r, b_l):
    x_p = jnp.concatenate([x, jnp.zeros((NP - N, F_IN), jnp.float32)])
    y, z = _dual_matmul(x_p, W_l.T, W_r.T)

    src = edge_index[0].astype(jnp.int32)
    dst = edge_index[1].astype(jnp.int32)
    pad = EP - E
    src_p = jnp.concatenate([src, jnp.zeros((pad,), jnp.int32)]).reshape(NW, CPW, CHUNK)
    dst_p = jnp.concatenate([dst, jnp.full((pad,), N, jnp.int32)]).reshape(NW, CPW, CHUNK)

    za = jnp.zeros((NP, H), jnp.float32)
    zc = jnp.zeros((NP, 16), jnp.float32)

    agg_p, cnt_p = _sc_aggregate(y, src_p, dst_p, za, zc)

    return _finish(agg_p, cnt_p, z, b_l.reshape(1, H))


# speedup vs baseline: 13.3001x; 1.1155x over previous
import functools

import jax
import jax.numpy as jnp
from jax import lax
from jax.experimental import pallas as pl
from jax.experimental.pallas import tpu as pltpu, tpu_sc as plsc

N = 10000
E = 320000
F_IN = 128
H = 64

NC = 2
NS = 16
NW = NC * NS
CHUNK = 128
EPW = E // NW
NFC = EPW // CHUNK
REM = EPW - NFC * CHUNK
NP = 10112
RPT = NP // NS


def _mm2_body(x_ref, wl_ref, wr_ref, y_ref, z_ref):
    xb = x_ref[...]
    dn = (((1,), (1,)), ((), ()))
    y_ref[...] = lax.dot_general(xb, wl_ref[...], dn,
                                 preferred_element_type=jnp.float32)
    z_ref[...] = lax.dot_general(xb, wr_ref[...], dn,
                                 preferred_element_type=jnp.float32)


def _dual_matmul(x, wl, wr):
    blk = 1000
    grid = N // blk
    return pl.pallas_call(
        _mm2_body,
        grid=(grid,),
        in_specs=[
            pl.BlockSpec((blk, F_IN), lambda i: (i, 0)),
            pl.BlockSpec((H, F_IN), lambda i: (0, 0)),
            pl.BlockSpec((H, F_IN), lambda i: (0, 0)),
        ],
        out_specs=[
            pl.BlockSpec((blk, H), lambda i: (i, 0)),
            pl.BlockSpec((blk, H), lambda i: (i, 0)),
        ],
        out_shape=[
            jax.ShapeDtypeStruct((NP, H), jnp.float32),
            jax.ShapeDtypeStruct((NP, H), jnp.float32),
        ],
    )(x, wl, wr)


def _sc_body(y_hbm, edge_hbm, za_hbm, zc_hbm,
             agg_out, cnt_out,
             y_sh, agg_sh, cnt_sh,
             src_v, dst_v, rows_v, ones_v, rows16_v,
             gsem, ssem, csem):
    c = lax.axis_index("c")
    s = lax.axis_index("s")
    w = c * NS + s

    pltpu.sync_copy(za_hbm.at[pl.ds(s * RPT, RPT)], agg_sh.at[pl.ds(s * RPT, RPT)])
    pltpu.sync_copy(zc_hbm.at[pl.ds(s * RPT, RPT)], cnt_sh.at[pl.ds(s * RPT, RPT)])
    pltpu.sync_copy(y_hbm.at[pl.ds(s * RPT, RPT)], y_sh.at[pl.ds(s * RPT, RPT)])

    pltpu.sync_copy(edge_hbm.at[0].at[pl.ds(w * EPW, EPW)], src_v)
    pltpu.sync_copy(edge_hbm.at[1].at[pl.ds(w * EPW, EPW)], dst_v)

    def ones_init(j, carry):
        ones_v[j] = jnp.ones((16,), jnp.float32)
        return carry

    lax.fori_loop(0, CHUNK, ones_init, 0)

    plsc.subcore_barrier()

    def sidx(j):
        return src_v.at[pl.ds(j * CHUNK, CHUNK)]

    def didx(j):
        return dst_v.at[pl.ds(j * CHUNK, CHUNK)]

    pltpu.async_copy(y_sh.at[sidx(0)], rows_v.at[0], gsem)

    def chunk_step(j, carry):
        b = lax.rem(j, 2)
        nb = 1 - b

        pltpu.make_async_copy(y_sh.at[sidx(j)], rows_v.at[b], gsem).wait()

        @pl.when(j >= 1)
        def _wait_prev_scatter():
            pltpu.make_async_copy(rows_v.at[nb], agg_sh.at[didx(j)],
                                  ssem).wait()
            pltpu.make_async_copy(ones_v, cnt_sh.at[didx(j)], csem).wait()

        @pl.when(j + 1 < NFC)
        def _prefetch_next():
            pltpu.async_copy(y_sh.at[sidx(j + 1)], rows_v.at[nb], gsem)

        pltpu.async_copy(rows_v.at[b], agg_sh.at[didx(j)], ssem, add=True)
        pltpu.async_copy(ones_v, cnt_sh.at[didx(j)], csem, add=True)
        return carry

    lax.fori_loop(0, NFC, chunk_step, 0)

    last = NFC - 1
    pltpu.make_async_copy(rows_v.at[lax.rem(last, 2)],
                          agg_sh.at[didx(last)], ssem).wait()
    pltpu.make_async_copy(ones_v, cnt_sh.at[didx(last)], csem).wait()

    rs = src_v.at[pl.ds(NFC * CHUNK, REM)]
    rd = dst_v.at[pl.ds(NFC * CHUNK, REM)]
    pltpu.sync_copy(y_sh.at[rs], rows16_v)
    pltpu.sync_copy(rows16_v, agg_sh.at[rd], add=True)
    pltpu.sync_copy(ones_v.at[pl.ds(0, REM)], cnt_sh.at[rd], add=True)

    plsc.subcore_barrier()

    pltpu.sync_copy(agg_sh.at[pl.ds(s * RPT, RPT)],
                    agg_out.at[c].at[pl.ds(s * RPT, RPT)])
    pltpu.sync_copy(cnt_sh.at[pl.ds(s * RPT, RPT)],
                    cnt_out.at[c].at[pl.ds(s * RPT, RPT)])


_sc_aggregate = pl.kernel(
    _sc_body,
    out_type=[
        jax.ShapeDtypeStruct((NC, NP, H), jnp.float32),
        jax.ShapeDtypeStruct((NC, NP, 16), jnp.float32),
    ],
    mesh=plsc.VectorSubcoreMesh(core_axis_name="c", subcore_axis_name="s"),
    compiler_params=pltpu.CompilerParams(use_tc_tiling_on_sc=False),
    scratch_types=[
        pltpu.VMEM_SHARED((NP, H), jnp.float32),
        pltpu.VMEM_SHARED((NP, H), jnp.float32),
        pltpu.VMEM_SHARED((NP, 16), jnp.float32),
        pltpu.VMEM((EPW,), jnp.int32),
        pltpu.VMEM((EPW,), jnp.int32),
        pltpu.VMEM((2, CHUNK, H), jnp.float32),
        pltpu.VMEM((CHUNK, 16), jnp.float32),
        pltpu.VMEM((REM, H), jnp.float32),
        pltpu.SemaphoreType.DMA,
        pltpu.SemaphoreType.DMA,
        pltpu.SemaphoreType.DMA,
    ],
)


def _fin_body(agg_ref, cnt_ref, z_ref, b_ref, o_ref):
    a = agg_ref[0] + agg_ref[1]
    cnt = jnp.maximum(cnt_ref[0][:, :1] + cnt_ref[1][:, :1], 1.0)
    o_ref[...] = a / cnt + z_ref[...] + b_ref[...]


def _finish(agg_p, cnt_p, z, b_row):
    blk = 1000
    grid = N // blk
    return pl.pallas_call(
        _fin_body,
        grid=(grid,),
        in_specs=[
            pl.BlockSpec((NC, blk, H), lambda i: (0, i, 0)),
            pl.BlockSpec((NC, blk, 16), lambda i: (0, i, 0)),
            pl.BlockSpec((blk, H), lambda i: (i, 0)),
            pl.BlockSpec((1, H), lambda i: (0, 0)),
        ],
        out_specs=pl.BlockSpec((blk, H), lambda i: (i, 0)),
        out_shape=jax.ShapeDtypeStruct((N, H), jnp.float32),
    )(agg_p, cnt_p, z, b_row)


def kernel(x, edge_index, W_l, W_r, b_l):
    y, z = _dual_matmul(x, W_l, W_r)

    za = jnp.zeros((NP, H), jnp.float32)
    zc = jnp.zeros((NP, 16), jnp.float32)

    agg_p, cnt_p = _sc_aggregate(y, edge_index.astype(jnp.int32), za, zc)

    return _finish(agg_p, cnt_p, z, b_l.reshape(1, H))


# speedup vs baseline: 17.3960x; 1.3080x over previous
import functools

import jax
import jax.numpy as jnp
from jax import lax
from jax.experimental import pallas as pl
from jax.experimental.pallas import tpu as pltpu, tpu_sc as plsc

N = 10000
E = 320000
F_IN = 128
H = 64

NC = 2
NS = 16
NW = NC * NS
CHUNK = 128
EPW = E // NW
NFC = EPW // CHUNK
REM = EPW - NFC * CHUNK
NP = 10112
RPT = NP // NS
NBUF = 4


def _mm2_body(x_ref, wl_ref, wr_ref, y_ref, z_ref):
    xb = x_ref[...]
    dn = (((1,), (1,)), ((), ()))
    y_ref[...] = lax.dot_general(xb, wl_ref[...], dn,
                                 preferred_element_type=jnp.float32)
    z_ref[...] = lax.dot_general(xb, wr_ref[...], dn,
                                 preferred_element_type=jnp.float32)


def _dual_matmul(x, wl, wr):
    blk = 1000
    grid = N // blk
    return pl.pallas_call(
        _mm2_body,
        grid=(grid,),
        in_specs=[
            pl.BlockSpec((blk, F_IN), lambda i: (i, 0)),
            pl.BlockSpec((H, F_IN), lambda i: (0, 0)),
            pl.BlockSpec((H, F_IN), lambda i: (0, 0)),
        ],
        out_specs=[
            pl.BlockSpec((blk, H), lambda i: (i, 0)),
            pl.BlockSpec((blk, H), lambda i: (i, 0)),
        ],
        out_shape=[
            jax.ShapeDtypeStruct((NP, H), jnp.float32),
            jax.ShapeDtypeStruct((NP, H), jnp.float32),
        ],
    )(x, wl, wr)


def _sc_body(y_hbm, edge_hbm, za_hbm, zc_hbm,
             agg_out, cnt_out,
             agg_sh, cnt_sh,
             src_v, dst_v, rows_v, ones_v, rows16_v,
             gsem, ssem, csem):
    c = lax.axis_index("c")
    s = lax.axis_index("s")
    w = c * NS + s

    pltpu.sync_copy(za_hbm.at[pl.ds(s * RPT, RPT)], agg_sh.at[pl.ds(s * RPT, RPT)])
    pltpu.sync_copy(zc_hbm.at[pl.ds(s * RPT, RPT)], cnt_sh.at[pl.ds(s * RPT, RPT)])

    pltpu.sync_copy(edge_hbm.at[0].at[pl.ds(w * EPW, EPW)], src_v)
    pltpu.sync_copy(edge_hbm.at[1].at[pl.ds(w * EPW, EPW)], dst_v)

    def ones_init(j, carry):
        ones_v[j] = jnp.ones((16,), jnp.float32)
        return carry

    lax.fori_loop(0, CHUNK, ones_init, 0)

    plsc.subcore_barrier()

    def sidx(j):
        return src_v.at[pl.ds(j * CHUNK, CHUNK)]

    def didx(j):
        return dst_v.at[pl.ds(j * CHUNK, CHUNK)]

    for p in range(NBUF - 1):
        pltpu.async_copy(y_hbm.at[sidx(p)], rows_v.at[p], gsem)

    def chunk_step(j, carry):
        b = lax.rem(j, NBUF)

        pltpu.make_async_copy(y_hbm.at[sidx(j)], rows_v.at[b], gsem).wait()

        @pl.when(j >= 1)
        def _wait_prev_scatter():
            pltpu.make_async_copy(rows_v.at[b], agg_sh.at[didx(j)],
                                  ssem).wait()
            pltpu.make_async_copy(ones_v, cnt_sh.at[didx(j)], csem).wait()

        @pl.when(j + NBUF - 1 < NFC)
        def _prefetch_next():
            pltpu.async_copy(y_hbm.at[sidx(j + NBUF - 1)],
                             rows_v.at[lax.rem(j + NBUF - 1, NBUF)], gsem)

        pltpu.async_copy(rows_v.at[b], agg_sh.at[didx(j)], ssem, add=True)
        pltpu.async_copy(ones_v, cnt_sh.at[didx(j)], csem, add=True)
        return carry

    lax.fori_loop(0, NFC, chunk_step, 0)

    last = NFC - 1
    pltpu.make_async_copy(rows_v.at[lax.rem(last, NBUF)],
                          agg_sh.at[didx(last)], ssem).wait()
    pltpu.make_async_copy(ones_v, cnt_sh.at[didx(last)], csem).wait()

    rs = src_v.at[pl.ds(NFC * CHUNK, REM)]
    rd = dst_v.at[pl.ds(NFC * CHUNK, REM)]
    pltpu.sync_copy(y_hbm.at[rs], rows16_v)
    pltpu.sync_copy(rows16_v, agg_sh.at[rd], add=True)
    pltpu.sync_copy(ones_v.at[pl.ds(0, REM)], cnt_sh.at[rd], add=True)

    plsc.subcore_barrier()

    pltpu.sync_copy(agg_sh.at[pl.ds(s * RPT, RPT)],
                    agg_out.at[c].at[pl.ds(s * RPT, RPT)])
    pltpu.sync_copy(cnt_sh.at[pl.ds(s * RPT, RPT)],
                    cnt_out.at[c].at[pl.ds(s * RPT, RPT)])


_sc_aggregate = pl.kernel(
    _sc_body,
    out_type=[
        jax.ShapeDtypeStruct((NC, NP, H), jnp.float32),
        jax.ShapeDtypeStruct((NC, NP, 16), jnp.float32),
    ],
    mesh=plsc.VectorSubcoreMesh(core_axis_name="c", subcore_axis_name="s"),
    compiler_params=pltpu.CompilerParams(use_tc_tiling_on_sc=False),
    scratch_types=[
        pltpu.VMEM_SHARED((NP, H), jnp.float32),
        pltpu.VMEM_SHARED((NP, 16), jnp.float32),
        pltpu.VMEM((EPW,), jnp.int32),
        pltpu.VMEM((EPW,), jnp.int32),
        pltpu.VMEM((NBUF, CHUNK, H), jnp.float32),
        pltpu.VMEM((CHUNK, 16), jnp.float32),
        pltpu.VMEM((REM, H), jnp.float32),
        pltpu.SemaphoreType.DMA,
        pltpu.SemaphoreType.DMA,
        pltpu.SemaphoreType.DMA,
    ],
)


def _fin_body(agg_ref, cnt_ref, z_ref, b_ref, o_ref):
    a = agg_ref[0] + agg_ref[1]
    cnt = jnp.maximum(cnt_ref[0][:, :1] + cnt_ref[1][:, :1], 1.0)
    o_ref[...] = a / cnt + z_ref[...] + b_ref[...]


def _finish(agg_p, cnt_p, z, b_row):
    blk = 1000
    grid = N // blk
    return pl.pallas_call(
        _fin_body,
        grid=(grid,),
        in_specs=[
            pl.BlockSpec((NC, blk, H), lambda i: (0, i, 0)),
            pl.BlockSpec((NC, blk, 16), lambda i: (0, i, 0)),
            pl.BlockSpec((blk, H), lambda i: (i, 0)),
            pl.BlockSpec((1, H), lambda i: (0, 0)),
        ],
        out_specs=pl.BlockSpec((blk, H), lambda i: (i, 0)),
        out_shape=jax.ShapeDtypeStruct((N, H), jnp.float32),
    )(agg_p, cnt_p, z, b_row)


def kernel(x, edge_index, W_l, W_r, b_l):
    y, z = _dual_matmul(x, W_l, W_r)

    za = jnp.zeros((NP, H), jnp.float32)
    zc = jnp.zeros((NP, 16), jnp.float32)

    agg_p, cnt_p = _sc_aggregate(y, edge_index.astype(jnp.int32), za, zc)

    return _finish(agg_p, cnt_p, z, b_l.reshape(1, H))
